# Initial kernel scaffold; baseline (speedup 1.0000x reference)
import jax
import jax.numpy as jnp
from jax.experimental import pallas as pl


def kernel(x, edge_index, v_node_x, gW0s, gW0n, gb0, gW1s, gW1n, gb1, gW2s, gW2n, gb2, mW1, mb1, mW2, mb2, hW1, hb1, hW2, hb2):
    raise NotImplementedError("write your pallas kernel here")


# speedup vs baseline: 4.7457x; 4.7457x over previous
import functools

import jax
import jax.numpy as jnp
from jax import lax
from jax.experimental import pallas as pl
from jax.experimental.pallas import tpu as pltpu
from jax.experimental.pallas import tpu_sc as plsc

N = 10000
E = 320000
D = 128
H = 64

NC = 2
NS = 16
NW = NC * NS
EPW = E // NW
CHUNK = 80
NCH = EPW // CHUNK
NPAD = 10240
RPT = NPAD // NS
DEG_PAD = NPAD
DPT = DEG_PAD // NS


def _seg_body(with_deg, *refs):
    if with_deg:
        (h_hbm, src_hbm, dst_hbm, znd_hbm, zdeg_hbm, ones_hbm,
         agg_out, deg_out, src_v, dst_v, rows_v, ones_v, agg_sh, deg_sh, sem) = refs
    else:
        (h_hbm, src_hbm, dst_hbm, znd_hbm,
         agg_out, src_v, dst_v, rows_v, agg_sh, sem) = refs
    c = lax.axis_index("c")
    s = lax.axis_index("s")
    w = s * NC + c

    t0 = s * RPT
    pltpu.sync_copy(znd_hbm.at[pl.ds(t0, RPT)], agg_sh.at[pl.ds(t0, RPT)])
    if with_deg:
        d0 = s * DPT
        pltpu.sync_copy(zdeg_hbm.at[pl.ds(d0, DPT)], deg_sh.at[pl.ds(d0, DPT)])
        pltpu.sync_copy(ones_hbm, ones_v)
    plsc.subcore_barrier()

    def chunk(j, carry):
        e0 = w * EPW + j * CHUNK
        pltpu.sync_copy(src_hbm.at[pl.ds(e0, CHUNK)], src_v)
        pltpu.sync_copy(dst_hbm.at[pl.ds(e0, CHUNK)], dst_v)
        pltpu.async_copy(h_hbm.at[src_v], rows_v, sem).wait()
        pltpu.sync_copy(rows_v, agg_sh.at[dst_v], add=True)
        if with_deg:
            pltpu.sync_copy(ones_v, deg_sh.at[dst_v], add=True)
        return carry

    lax.fori_loop(0, NCH, chunk, 0)
    plsc.subcore_barrier()

    pltpu.sync_copy(agg_sh.at[pl.ds(t0, RPT)], agg_out.at[c, pl.ds(t0, RPT)])
    if with_deg:
        pltpu.sync_copy(deg_sh.at[pl.ds(d0, DPT)], deg_out.at[c, pl.ds(d0, DPT)])


def _make_seg(with_deg):
    mesh = plsc.VectorSubcoreMesh(core_axis_name="c", subcore_axis_name="s")
    out_type = [jax.ShapeDtypeStruct((NC, NPAD, D), jnp.float32)]
    if with_deg:
        out_type.append(jax.ShapeDtypeStruct((NC, DEG_PAD), jnp.float32))
    scratch = [
        pltpu.VMEM((CHUNK,), jnp.int32),
        pltpu.VMEM((CHUNK,), jnp.int32),
        pltpu.VMEM((CHUNK, D), jnp.float32),
    ]
    if with_deg:
        scratch.append(pltpu.VMEM((CHUNK,), jnp.float32))
    scratch += [
        pltpu.VMEM_SHARED((NPAD, D), jnp.float32),
    ]
    if with_deg:
        scratch.append(pltpu.VMEM_SHARED((DEG_PAD,), jnp.float32))
    scratch.append(pltpu.SemaphoreType.DMA)
    return pl.kernel(
        functools.partial(_seg_body, with_deg),
        out_type=out_type,
        mesh=mesh,
        scratch_types=scratch,
    )


_seg_deg = _make_seg(True)
_seg = _make_seg(False)

BLK = 1000
GRID = N // BLK


def _layer_body(h_ref, p_ref, dd_ref, ws_ref, wn_ref, b_ref, o_ref, *, relu):
    agg = p_ref[0] + p_ref[1]
    deg = dd_ref[0] + dd_ref[1]
    rdeg = 1.0 / jnp.maximum(deg, 1.0)
    out = (jnp.dot(h_ref[...], ws_ref[...], preferred_element_type=jnp.float32)
           + jnp.dot(agg * rdeg, wn_ref[...], preferred_element_type=jnp.float32)
           + b_ref[...])
    o_ref[...] = jnp.maximum(out, 0.0) if relu else out


def _head_body(h_ref, p_ref, dd_ref, ws_ref, wn_ref, b_ref, v_ref,
               mw1_ref, mb1_ref, mw2_ref, mb2_ref,
               hw1_ref, hb1_ref, hw2_ref, hb2_ref, o_ref):
    agg = p_ref[0] + p_ref[1]
    deg = dd_ref[0] + dd_ref[1]
    rdeg = 1.0 / jnp.maximum(deg, 1.0)
    h3 = (jnp.dot(h_ref[...], ws_ref[...], preferred_element_type=jnp.float32)
          + jnp.dot(agg * rdeg, wn_ref[...], preferred_element_type=jnp.float32)
          + b_ref[...])
    v = jnp.maximum(
        jnp.dot(v_ref[...], mw1_ref[...], preferred_element_type=jnp.float32)
        + mb1_ref[...], 0.0)
    v = jnp.dot(v, mw2_ref[...], preferred_element_type=jnp.float32) + mb2_ref[...]
    f = h3 + v
    o = jnp.maximum(
        jnp.dot(f, hw1_ref[...], preferred_element_type=jnp.float32)
        + hb1_ref[...], 0.0)
    o_ref[...] = (jnp.dot(o, hw2_ref[...], preferred_element_type=jnp.float32)
                  + hb2_ref[...])


def _full(shape):
    return pl.BlockSpec(shape, lambda i: tuple(0 for _ in shape))


_SPEC_H = pl.BlockSpec((BLK, D), lambda i: (i, 0))
_SPEC_P = pl.BlockSpec((NC, BLK, D), lambda i: (0, i, 0))
_SPEC_DD = pl.BlockSpec((NC, BLK, 1), lambda i: (0, i, 0))


def _tc_layer(h, p, dd, Ws, Wn, b, relu):
    return pl.pallas_call(
        functools.partial(_layer_body, relu=relu),
        grid=(GRID,),
        in_specs=[_SPEC_H, _SPEC_P, _SPEC_DD,
                  _full((D, D)), _full((D, D)), _full((1, D))],
        out_specs=_SPEC_H,
        out_shape=jax.ShapeDtypeStruct((N, D), jnp.float32),
    )(h, p, dd, Ws, Wn, b)


def _tc_head(h, p, dd, Ws, Wn, b, v, mW1, mb1, mW2, mb2, hW1, hb1, hW2, hb2):
    return pl.pallas_call(
        _head_body,
        grid=(GRID,),
        in_specs=[_SPEC_H, _SPEC_P, _SPEC_DD,
                  _full((D, D)), _full((D, D)), _full((1, D)),
                  _full((1, D)),
                  _full((D, D)), _full((1, D)), _full((D, D)), _full((1, D)),
                  _full((D, H)), _full((1, H)), _full((H, 1)), _full((1, 1))],
        out_specs=pl.BlockSpec((BLK, 1), lambda i: (i, 0)),
        out_shape=jax.ShapeDtypeStruct((N, 1), jnp.float32),
    )(h, p, dd, Ws, Wn, b, v, mW1, mb1, mW2, mb2, hW1, hb1, hW2, hb2)


def kernel(x, edge_index, v_node_x, gW0s, gW0n, gb0, gW1s, gW1n, gb1,
           gW2s, gW2n, gb2, mW1, mb1, mW2, mb2, hW1, hb1, hW2, hb2):
    src = edge_index[0]
    dst = edge_index[1]
    znd = jnp.zeros((NPAD, D), jnp.float32)
    zdeg = jnp.zeros((DEG_PAD,), jnp.float32)
    ones = jnp.ones((CHUNK,), jnp.float32)

    agg0, degp = _seg_deg(x, src, dst, znd, zdeg, ones)
    dd = degp.reshape(NC, DEG_PAD, 1)

    b0 = gb0.reshape(1, D)
    b1 = gb1.reshape(1, D)
    b2 = gb2.reshape(1, D)

    h1 = _tc_layer(x, agg0, dd, gW0s, gW0n, b0, True)
    (agg1,) = _seg(h1, src, dst, znd)
    h2 = _tc_layer(h1, agg1, dd, gW1s, gW1n, b1, True)
    (agg2,) = _seg(h2, src, dst, znd)
    out = _tc_head(h2, agg2, dd, gW2s, gW2n, b2, v_node_x,
                   mW1, mb1.reshape(1, D), mW2, mb2.reshape(1, D),
                   hW1, hb1.reshape(1, H), hW2, hb2.reshape(1, 1))
    return out.reshape(1, N)


# speedup vs baseline: 8.7974x; 1.8537x over previous
import functools

import jax
import jax.numpy as jnp
from jax import lax
from jax.experimental import pallas as pl
from jax.experimental.pallas import tpu as pltpu
from jax.experimental.pallas import tpu_sc as plsc

N = 10000
E = 320000
D = 128
H = 64

NC = 2
NS = 16
NW = NC * NS
EPW = E // NW
CHUNK = 128
NPAD = 10240
PAD_ROWS = NPAD - N
EPT = 10240
PAD_E = EPT - EPW
NCH = EPT // CHUNK
RPT = NPAD // NS
DEG_PAD = NPAD
DPT = DEG_PAD // NS


def _seg_body(with_deg, *refs):
    if with_deg:
        (h_hbm, si_hbm, znd_hbm, zdeg_hbm, ones_hbm,
         agg_out, deg_out, ia, ib, rows_a, rows_b, ones_v,
         agg_sh, deg_sh, sem_a, sem_b, sem_ia, sem_ib) = refs
    else:
        (h_hbm, si_hbm, znd_hbm,
         agg_out, ia, ib, rows_a, rows_b, agg_sh,
         sem_a, sem_b, sem_ia, sem_ib) = refs
    c = lax.axis_index("c")
    s = lax.axis_index("s")
    w = s * NC + c

    t0 = s * RPT
    pltpu.sync_copy(znd_hbm.at[pl.ds(t0, RPT)], agg_sh.at[pl.ds(t0, RPT)])
    if with_deg:
        d0 = s * DPT
        pltpu.sync_copy(zdeg_hbm.at[pl.ds(d0, DPT)], deg_sh.at[pl.ds(d0, DPT)])
        pltpu.sync_copy(ones_hbm, ones_v)
    pltpu.sync_copy(si_hbm.at[w, 0], ia)

    idummy = si_hbm.at[0, 0]
    rdummy = znd_hbm.at[pl.ds(0, CHUNK)]

    def igather(j, buf, sem):
        pltpu.async_copy(si_hbm.at[w, j], buf, sem)

    def gather(ibuf, buf, sem):
        pltpu.async_copy(h_hbm.at[ibuf.at[0]], buf, sem)

    def scat(ibuf, buf):
        pltpu.sync_copy(buf, agg_sh.at[ibuf.at[1]], add=True)
        if with_deg:
            pltpu.sync_copy(ones_v, deg_sh.at[ibuf.at[1]], add=True)

    gather(ia, rows_a, sem_a)
    igather(1, ib, sem_ib)
    plsc.subcore_barrier()

    def pair(jj, carry):
        j0 = 2 * jj
        pltpu.make_async_copy(rdummy, rows_a, sem_a).wait()
        pltpu.make_async_copy(idummy, ib, sem_ib).wait()
        gather(ib, rows_b, sem_b)
        scat(ia, rows_a)
        igather(j0 + 2, ia, sem_ia)
        pltpu.make_async_copy(rdummy, rows_b, sem_b).wait()
        scat(ib, rows_b)
        pltpu.make_async_copy(idummy, ia, sem_ia).wait()
        gather(ia, rows_a, sem_a)
        igather(j0 + 3, ib, sem_ib)
        return carry

    lax.fori_loop(0, NCH // 2 - 1, pair, 0)
    pltpu.make_async_copy(rdummy, rows_a, sem_a).wait()
    pltpu.make_async_copy(idummy, ib, sem_ib).wait()
    gather(ib, rows_b, sem_b)
    scat(ia, rows_a)
    pltpu.make_async_copy(rdummy, rows_b, sem_b).wait()
    scat(ib, rows_b)
    plsc.subcore_barrier()

    pltpu.sync_copy(agg_sh.at[pl.ds(t0, RPT)], agg_out.at[c, pl.ds(t0, RPT)])
    if with_deg:
        pltpu.sync_copy(deg_sh.at[pl.ds(d0, DPT)], deg_out.at[c, pl.ds(d0, DPT)])


def _make_seg(with_deg):
    mesh = plsc.VectorSubcoreMesh(core_axis_name="c", subcore_axis_name="s")
    out_type = [jax.ShapeDtypeStruct((NC, NPAD, D), jnp.float32)]
    if with_deg:
        out_type.append(jax.ShapeDtypeStruct((NC, DEG_PAD), jnp.float32))
    scratch = [
        pltpu.VMEM((2, CHUNK), jnp.int32),
        pltpu.VMEM((2, CHUNK), jnp.int32),
        pltpu.VMEM((CHUNK, D), jnp.float32),
        pltpu.VMEM((CHUNK, D), jnp.float32),
    ]
    if with_deg:
        scratch.append(pltpu.VMEM((CHUNK,), jnp.float32))
    scratch += [
        pltpu.VMEM_SHARED((NPAD, D), jnp.float32),
    ]
    if with_deg:
        scratch.append(pltpu.VMEM_SHARED((DEG_PAD,), jnp.float32))
    scratch += [pltpu.SemaphoreType.DMA, pltpu.SemaphoreType.DMA,
                pltpu.SemaphoreType.DMA, pltpu.SemaphoreType.DMA]
    return pl.kernel(
        functools.partial(_seg_body, with_deg),
        out_type=out_type,
        mesh=mesh,
        scratch_types=scratch,
    )


_seg_deg = _make_seg(True)
_seg = _make_seg(False)

BLK = 1000
GRID = N // BLK


def _layer_body(h_ref, p_ref, dd_ref, ws_ref, wn_ref, b_ref, o_ref, *, relu):
    agg = p_ref[0] + p_ref[1]
    deg = dd_ref[0] + dd_ref[1]
    rdeg = 1.0 / jnp.maximum(deg, 1.0)
    out = (jnp.dot(h_ref[...], ws_ref[...], preferred_element_type=jnp.float32)
           + jnp.dot(agg * rdeg, wn_ref[...], preferred_element_type=jnp.float32)
           + b_ref[...])
    o_ref[...] = jnp.maximum(out, 0.0) if relu else out


def _head_body(h_ref, p_ref, dd_ref, ws_ref, wn_ref, b_ref, v_ref,
               mw1_ref, mb1_ref, mw2_ref, mb2_ref,
               hw1_ref, hb1_ref, hw2_ref, hb2_ref, o_ref):
    agg = p_ref[0] + p_ref[1]
    deg = dd_ref[0] + dd_ref[1]
    rdeg = 1.0 / jnp.maximum(deg, 1.0)
    h3 = (jnp.dot(h_ref[...], ws_ref[...], preferred_element_type=jnp.float32)
          + jnp.dot(agg * rdeg, wn_ref[...], preferred_element_type=jnp.float32)
          + b_ref[...])
    v = jnp.maximum(
        jnp.dot(v_ref[...], mw1_ref[...], preferred_element_type=jnp.float32)
        + mb1_ref[...], 0.0)
    v = jnp.dot(v, mw2_ref[...], preferred_element_type=jnp.float32) + mb2_ref[...]
    f = h3 + v
    o = jnp.maximum(
        jnp.dot(f, hw1_ref[...], preferred_element_type=jnp.float32)
        + hb1_ref[...], 0.0)
    o_ref[...] = (jnp.dot(o, hw2_ref[...], preferred_element_type=jnp.float32)
                  + hb2_ref[...])


def _full(shape):
    return pl.BlockSpec(shape, lambda i: tuple(0 for _ in shape))


_SPEC_H = pl.BlockSpec((BLK, D), lambda i: (i, 0))
_SPEC_P = pl.BlockSpec((NC, BLK, D), lambda i: (0, i, 0))
_SPEC_DD = pl.BlockSpec((NC, BLK, 1), lambda i: (0, i, 0))


def _tc_layer(h, p, dd, Ws, Wn, b, relu):
    return pl.pallas_call(
        functools.partial(_layer_body, relu=relu),
        grid=(GRID,),
        in_specs=[_SPEC_H, _SPEC_P, _SPEC_DD,
                  _full((D, D)), _full((D, D)), _full((1, D))],
        out_specs=_SPEC_H,
        out_shape=jax.ShapeDtypeStruct((N, D), jnp.float32),
    )(h, p, dd, Ws, Wn, b)


def _tc_head(h, p, dd, Ws, Wn, b, v, mW1, mb1, mW2, mb2, hW1, hb1, hW2, hb2):
    return pl.pallas_call(
        _head_body,
        grid=(GRID,),
        in_specs=[_SPEC_H, _SPEC_P, _SPEC_DD,
                  _full((D, D)), _full((D, D)), _full((1, D)),
                  _full((1, D)),
                  _full((D, D)), _full((1, D)), _full((D, D)), _full((1, D)),
                  _full((D, H)), _full((1, H)), _full((H, 1)), _full((1, 1))],
        out_specs=pl.BlockSpec((BLK, 1), lambda i: (i, 0)),
        out_shape=jax.ShapeDtypeStruct((N, 1), jnp.float32),
    )(h, p, dd, Ws, Wn, b, v, mW1, mb1, mW2, mb2, hW1, hb1, hW2, hb2)


def kernel(x, edge_index, v_node_x, gW0s, gW0n, gb0, gW1s, gW1n, gb1,
           gW2s, gW2n, gb2, mW1, mb1, mW2, mb2, hW1, hb1, hW2, hb2):
    wid = jnp.arange(NW, dtype=jnp.int32)[:, None]
    pad = jnp.arange(PAD_E, dtype=jnp.int32)[None, :]
    pad_src = (pad * 41 + wid * 313) % N
    pad_dst = N + (pad + wid * 7) % PAD_ROWS
    src3 = jnp.concatenate(
        [edge_index[0].reshape(NW, EPW), pad_src], axis=1).reshape(NW, NCH, CHUNK)
    dst3 = jnp.concatenate(
        [edge_index[1].reshape(NW, EPW), pad_dst], axis=1).reshape(NW, NCH, CHUNK)
    si = jnp.stack([src3, dst3], axis=2)
    znd = jnp.zeros((NPAD, D), jnp.float32)
    zdeg = jnp.zeros((DEG_PAD,), jnp.float32)
    ones = jnp.ones((CHUNK,), jnp.float32)

    agg0, degp = _seg_deg(x, si, znd, zdeg, ones)
    dd = degp.reshape(NC, DEG_PAD, 1)

    b0 = gb0.reshape(1, D)
    b1 = gb1.reshape(1, D)
    b2 = gb2.reshape(1, D)

    h1 = _tc_layer(x, agg0, dd, gW0s, gW0n, b0, True)
    (agg1,) = _seg(h1, si, znd)
    h2 = _tc_layer(h1, agg1, dd, gW1s, gW1n, b1, True)
    (agg2,) = _seg(h2, si, znd)
    out = _tc_head(h2, agg2, dd, gW2s, gW2n, b2, v_node_x,
                   mW1, mb1.reshape(1, D), mW2, mb2.reshape(1, D),
                   hW1, hb1.reshape(1, H), hW2, hb2.reshape(1, 1))
    return out.reshape(1, N)


# speedup vs baseline: 9.9030x; 1.1257x over previous
import functools

import jax
import jax.numpy as jnp
from jax import lax
from jax.experimental import pallas as pl
from jax.experimental.pallas import tpu as pltpu
from jax.experimental.pallas import tpu_sc as plsc

N = 10000
E = 320000
D = 128
H = 64

NC = 2
NS = 16
NW = NC * NS
EPW = E // NW
CHUNK = 128
NPAD = 10240
PAD_ROWS = NPAD - N
EPT = 10240
PAD_E = EPT - EPW
NCH = EPT // CHUNK
RPT = NPAD // NS
DEG_PAD = NPAD
DPT = DEG_PAD // NS


def _seg_body(with_deg, *refs):
    if with_deg:
        (h_hbm, si_hbm, znd_hbm, zdeg_hbm, ones_hbm,
         agg_out, deg_out, i0, i1, i2, i3, rows_a, rows_b, ones_v,
         agg_sh, deg_sh, sg0, sg1, ss0, ss1, si0, si1, si2, si3) = refs
    else:
        (h_hbm, si_hbm, znd_hbm,
         agg_out, i0, i1, i2, i3, rows_a, rows_b, agg_sh,
         sg0, sg1, ss0, ss1, si0, si1, si2, si3) = refs
    c = lax.axis_index("c")
    s = lax.axis_index("s")
    w = s * NC + c

    t0 = s * RPT
    pltpu.sync_copy(znd_hbm.at[pl.ds(t0, RPT)], agg_sh.at[pl.ds(t0, RPT)])
    if with_deg:
        d0 = s * DPT
        pltpu.sync_copy(zdeg_hbm.at[pl.ds(d0, DPT)], deg_sh.at[pl.ds(d0, DPT)])
        pltpu.sync_copy(ones_hbm, ones_v)

    I = [i0, i1, i2, i3]
    R = [rows_a, rows_b]
    GS = [sg0, sg1]
    SS = [ss0, ss1]
    IS = [si0, si1, si2, si3]
    idummy = si_hbm.at[0, 0]
    rdummy = znd_hbm.at[pl.ds(0, CHUNK)]

    def ifetch(j, k):
        pltpu.async_copy(si_hbm.at[w, jnp.minimum(j, NCH - 1)], I[k], IS[k])

    def slot(q, j, skip_swait=False):
        p, pn = q % 2, 1 - q % 2
        k, kp, kf = q % 4, (q + 1) % 4, (q + 3) % 4
        pltpu.make_async_copy(rdummy, R[p], GS[p]).wait()
        pltpu.async_copy(R[p], agg_sh.at[I[k].at[1]], SS[p], add=True)
        if with_deg:
            pltpu.async_copy(ones_v, deg_sh.at[I[k].at[1]], SS[p], add=True)
        if not skip_swait:
            pltpu.make_async_copy(rdummy, R[pn], SS[pn]).wait()
            if with_deg:
                pltpu.make_async_copy(ones_hbm, ones_v, SS[pn]).wait()
        ifetch(j + 3, kf)
        pltpu.make_async_copy(idummy, I[kp], IS[kp]).wait()
        pltpu.async_copy(h_hbm.at[I[kp].at[0]],
                         R[pn], GS[pn])

    pltpu.sync_copy(si_hbm.at[w, 0], i0)
    ifetch(1, 1)
    ifetch(2, 2)
    pltpu.async_copy(h_hbm.at[i0.at[0]], rows_a, sg0)
    plsc.subcore_barrier()

    slot(0, 0, skip_swait=True)
    slot(1, 1)
    slot(2, 2)
    slot(3, 3)

    def group(t, carry):
        j = 4 * t
        slot(0, j)
        slot(1, j + 1)
        slot(2, j + 2)
        slot(3, j + 3)
        return carry

    lax.fori_loop(1, NCH // 4, group, 0)

    pltpu.make_async_copy(rdummy, rows_a, sg0).wait()
    pltpu.make_async_copy(rdummy, rows_b, ss1).wait()
    if with_deg:
        pltpu.make_async_copy(ones_hbm, ones_v, ss1).wait()
    pltpu.make_async_copy(idummy, i1, si1).wait()
    pltpu.make_async_copy(idummy, i2, si2).wait()
    plsc.subcore_barrier()

    pltpu.sync_copy(agg_sh.at[pl.ds(t0, RPT)], agg_out.at[c, pl.ds(t0, RPT)])
    if with_deg:
        pltpu.sync_copy(deg_sh.at[pl.ds(d0, DPT)], deg_out.at[c, pl.ds(d0, DPT)])


def _make_seg(with_deg):
    mesh = plsc.VectorSubcoreMesh(core_axis_name="c", subcore_axis_name="s")
    out_type = [jax.ShapeDtypeStruct((NC, NPAD, D), jnp.float32)]
    if with_deg:
        out_type.append(jax.ShapeDtypeStruct((NC, DEG_PAD), jnp.float32))
    scratch = [
        pltpu.VMEM((2, CHUNK), jnp.int32),
        pltpu.VMEM((2, CHUNK), jnp.int32),
        pltpu.VMEM((2, CHUNK), jnp.int32),
        pltpu.VMEM((2, CHUNK), jnp.int32),
        pltpu.VMEM((CHUNK, D), jnp.float32),
        pltpu.VMEM((CHUNK, D), jnp.float32),
    ]
    if with_deg:
        scratch.append(pltpu.VMEM((CHUNK,), jnp.float32))
    scratch += [
        pltpu.VMEM_SHARED((NPAD, D), jnp.float32),
    ]
    if with_deg:
        scratch.append(pltpu.VMEM_SHARED((DEG_PAD,), jnp.float32))
    scratch += [pltpu.SemaphoreType.DMA] * 8
    return pl.kernel(
        functools.partial(_seg_body, with_deg),
        out_type=out_type,
        mesh=mesh,
        scratch_types=scratch,
    )


_seg_deg = _make_seg(True)
_seg = _make_seg(False)

BLK = 1000
GRID = N // BLK


def _layer_body(h_ref, p_ref, dd_ref, ws_ref, wn_ref, b_ref, o_ref, *, relu):
    agg = p_ref[0] + p_ref[1]
    deg = dd_ref[0] + dd_ref[1]
    rdeg = 1.0 / jnp.maximum(deg, 1.0)
    out = (jnp.dot(h_ref[...], ws_ref[...], preferred_element_type=jnp.float32)
           + jnp.dot(agg * rdeg, wn_ref[...], preferred_element_type=jnp.float32)
           + b_ref[...])
    o_ref[...] = jnp.maximum(out, 0.0) if relu else out


def _head_body(h_ref, p_ref, dd_ref, ws_ref, wn_ref, b_ref, v_ref,
               mw1_ref, mb1_ref, mw2_ref, mb2_ref,
               hw1_ref, hb1_ref, hw2_ref, hb2_ref, o_ref):
    agg = p_ref[0] + p_ref[1]
    deg = dd_ref[0] + dd_ref[1]
    rdeg = 1.0 / jnp.maximum(deg, 1.0)
    h3 = (jnp.dot(h_ref[...], ws_ref[...], preferred_element_type=jnp.float32)
          + jnp.dot(agg * rdeg, wn_ref[...], preferred_element_type=jnp.float32)
          + b_ref[...])
    v = jnp.maximum(
        jnp.dot(v_ref[...], mw1_ref[...], preferred_element_type=jnp.float32)
        + mb1_ref[...], 0.0)
    v = jnp.dot(v, mw2_ref[...], preferred_element_type=jnp.float32) + mb2_ref[...]
    f = h3 + v
    o = jnp.maximum(
        jnp.dot(f, hw1_ref[...], preferred_element_type=jnp.float32)
        + hb1_ref[...], 0.0)
    o_ref[...] = (jnp.dot(o, hw2_ref[...], preferred_element_type=jnp.float32)
                  + hb2_ref[...])


def _full(shape):
    return pl.BlockSpec(shape, lambda i: tuple(0 for _ in shape))


_SPEC_H = pl.BlockSpec((BLK, D), lambda i: (i, 0))
_SPEC_P = pl.BlockSpec((NC, BLK, D), lambda i: (0, i, 0))
_SPEC_DD = pl.BlockSpec((NC, BLK, 1), lambda i: (0, i, 0))


def _tc_layer(h, p, dd, Ws, Wn, b, relu):
    return pl.pallas_call(
        functools.partial(_layer_body, relu=relu),
        grid=(GRID,),
        in_specs=[_SPEC_H, _SPEC_P, _SPEC_DD,
                  _full((D, D)), _full((D, D)), _full((1, D))],
        out_specs=_SPEC_H,
        out_shape=jax.ShapeDtypeStruct((N, D), jnp.float32),
    )(h, p, dd, Ws, Wn, b)


def _tc_head(h, p, dd, Ws, Wn, b, v, mW1, mb1, mW2, mb2, hW1, hb1, hW2, hb2):
    return pl.pallas_call(
        _head_body,
        grid=(GRID,),
        in_specs=[_SPEC_H, _SPEC_P, _SPEC_DD,
                  _full((D, D)), _full((D, D)), _full((1, D)),
                  _full((1, D)),
                  _full((D, D)), _full((1, D)), _full((D, D)), _full((1, D)),
                  _full((D, H)), _full((1, H)), _full((H, 1)), _full((1, 1))],
        out_specs=pl.BlockSpec((BLK, 1), lambda i: (i, 0)),
        out_shape=jax.ShapeDtypeStruct((N, 1), jnp.float32),
    )(h, p, dd, Ws, Wn, b, v, mW1, mb1, mW2, mb2, hW1, hb1, hW2, hb2)


def kernel(x, edge_index, v_node_x, gW0s, gW0n, gb0, gW1s, gW1n, gb1,
           gW2s, gW2n, gb2, mW1, mb1, mW2, mb2, hW1, hb1, hW2, hb2):
    wid = jnp.arange(NW, dtype=jnp.int32)[:, None]
    pad = jnp.arange(PAD_E, dtype=jnp.int32)[None, :]
    pad_src = (pad * 41 + wid * 313) % N
    pad_dst = N + (pad + wid * 7) % PAD_ROWS
    src3 = jnp.concatenate(
        [edge_index[0].reshape(NW, EPW), pad_src], axis=1).reshape(NW, NCH, CHUNK)
    dst3 = jnp.concatenate(
        [edge_index[1].reshape(NW, EPW), pad_dst], axis=1).reshape(NW, NCH, CHUNK)
    si = jnp.stack([src3, dst3], axis=2)
    znd = jnp.zeros((NPAD, D), jnp.float32)
    zdeg = jnp.zeros((DEG_PAD,), jnp.float32)
    ones = jnp.ones((CHUNK,), jnp.float32)

    agg0, degp = _seg_deg(x, si, znd, zdeg, ones)
    dd = degp.reshape(NC, DEG_PAD, 1)

    b0 = gb0.reshape(1, D)
    b1 = gb1.reshape(1, D)
    b2 = gb2.reshape(1, D)

    h1 = _tc_layer(x, agg0, dd, gW0s, gW0n, b0, True)
    (agg1,) = _seg(h1, si, znd)
    h2 = _tc_layer(h1, agg1, dd, gW1s, gW1n, b1, True)
    (agg2,) = _seg(h2, si, znd)
    out = _tc_head(h2, agg2, dd, gW2s, gW2n, b2, v_node_x,
                   mW1, mb1.reshape(1, D), mW2, mb2.reshape(1, D),
                   hW1, hb1.reshape(1, H), hW2, hb2.reshape(1, 1))
    return out.reshape(1, N)


# speedup vs baseline: 10.4486x; 1.0551x over previous
import functools

import jax
import jax.numpy as jnp
from jax import lax
from jax.experimental import pallas as pl
from jax.experimental.pallas import tpu as pltpu
from jax.experimental.pallas import tpu_sc as plsc

N = 10000
E = 320000
D = 128
H = 64

NC = 2
NS = 16
NW = NC * NS
CHUNK = 128
NCHT = E // CHUNK
NSLOT = NCHT // NW
LEFT = NCHT - NSLOT * NW
NPAD = 10240
RPT = NPAD // NS
DEG_PAD = NPAD
DPT = DEG_PAD // NS


def _seg_body(with_deg, *refs):
    if with_deg:
        (h_hbm, ei_hbm, znd_hbm, zdeg_hbm, ones_hbm,
         agg_out, deg_out, i0, i1, i2, i3, rows_a, rows_b, ones_v,
         agg_sh, deg_sh, sg0, sg1, ss0, ss1, si0, si1, si2, si3) = refs
    else:
        (h_hbm, ei_hbm, znd_hbm,
         agg_out, i0, i1, i2, i3, rows_a, rows_b, agg_sh,
         sg0, sg1, ss0, ss1, si0, si1, si2, si3) = refs
    c = lax.axis_index("c")
    s = lax.axis_index("s")
    w = s * NC + c

    t0 = s * RPT
    pltpu.sync_copy(znd_hbm.at[pl.ds(t0, RPT)], agg_sh.at[pl.ds(t0, RPT)])
    if with_deg:
        d0 = s * DPT
        pltpu.sync_copy(zdeg_hbm.at[pl.ds(d0, DPT)], deg_sh.at[pl.ds(d0, DPT)])
        pltpu.sync_copy(ones_hbm, ones_v)

    I = [i0, i1, i2, i3]
    R = [rows_a, rows_b]
    GS = [sg0, sg1]
    SS = [ss0, ss1]
    IS = [si0, si1, si2, si3]
    idummy = ei_hbm.at[:, pl.ds(0, CHUNK)]
    rdummy = znd_hbm.at[pl.ds(0, CHUNK)]

    def ifetch(j, k):
        e0 = jnp.minimum(j * NW + w, NCHT - 1) * CHUNK
        pltpu.async_copy(ei_hbm.at[:, pl.ds(e0, CHUNK)], I[k], IS[k])

    def slot(q, j, skip_swait=False):
        p, pn = q % 2, 1 - q % 2
        k, kp, kf = q % 4, (q + 1) % 4, (q + 3) % 4
        pltpu.make_async_copy(rdummy, R[p], GS[p]).wait()
        pltpu.async_copy(R[p], agg_sh.at[I[k].at[1]], SS[p], add=True)
        if with_deg:
            pltpu.async_copy(ones_v, deg_sh.at[I[k].at[1]], SS[p], add=True)
        if not skip_swait:
            pltpu.make_async_copy(rdummy, R[pn], SS[pn]).wait()
            if with_deg:
                pltpu.make_async_copy(ones_hbm, ones_v, SS[pn]).wait()
        ifetch(j + 3, kf)
        pltpu.make_async_copy(idummy, I[kp], IS[kp]).wait()
        pltpu.async_copy(h_hbm.at[I[kp].at[0]],
                         R[pn], GS[pn])

    pltpu.sync_copy(ei_hbm.at[:, pl.ds(w * CHUNK, CHUNK)], i0)
    ifetch(1, 1)
    ifetch(2, 2)
    pltpu.async_copy(h_hbm.at[i0.at[0]], rows_a, sg0)
    plsc.subcore_barrier()

    slot(0, 0, skip_swait=True)
    slot(1, 1)
    slot(2, 2)
    slot(3, 3)

    def group(t, carry):
        j = 4 * t
        slot(0, j)
        slot(1, j + 1)
        slot(2, j + 2)
        slot(3, j + 3)
        return carry

    lax.fori_loop(1, NSLOT // 4, group, 0)
    slot(0, NSLOT - 2)
    slot(1, NSLOT - 1)

    pltpu.make_async_copy(rdummy, rows_a, sg0).wait()

    @pl.when(w < LEFT)
    def _leftover():
        pltpu.async_copy(rows_a, agg_sh.at[i2.at[1]], ss0, add=True)
        if with_deg:
            pltpu.async_copy(ones_v, deg_sh.at[i2.at[1]], ss0, add=True)

    pltpu.make_async_copy(rdummy, rows_b, ss1).wait()
    if with_deg:
        pltpu.make_async_copy(ones_hbm, ones_v, ss1).wait()

    @pl.when(w < LEFT)
    def _drain_leftover():
        pltpu.make_async_copy(rdummy, rows_a, ss0).wait()
        if with_deg:
            pltpu.make_async_copy(ones_hbm, ones_v, ss0).wait()

    pltpu.make_async_copy(idummy, i3, si3).wait()
    pltpu.make_async_copy(idummy, i0, si0).wait()
    plsc.subcore_barrier()

    pltpu.sync_copy(agg_sh.at[pl.ds(t0, RPT)], agg_out.at[c, pl.ds(t0, RPT)])
    if with_deg:
        pltpu.sync_copy(deg_sh.at[pl.ds(d0, DPT)], deg_out.at[c, pl.ds(d0, DPT)])


def _make_seg(with_deg):
    mesh = plsc.VectorSubcoreMesh(core_axis_name="c", subcore_axis_name="s")
    out_type = [jax.ShapeDtypeStruct((NC, NPAD, D), jnp.float32)]
    if with_deg:
        out_type.append(jax.ShapeDtypeStruct((NC, DEG_PAD), jnp.float32))
    scratch = [
        pltpu.VMEM((2, CHUNK), jnp.int32),
        pltpu.VMEM((2, CHUNK), jnp.int32),
        pltpu.VMEM((2, CHUNK), jnp.int32),
        pltpu.VMEM((2, CHUNK), jnp.int32),
        pltpu.VMEM((CHUNK, D), jnp.float32),
        pltpu.VMEM((CHUNK, D), jnp.float32),
    ]
    if with_deg:
        scratch.append(pltpu.VMEM((CHUNK,), jnp.float32))
    scratch += [
        pltpu.VMEM_SHARED((NPAD, D), jnp.float32),
    ]
    if with_deg:
        scratch.append(pltpu.VMEM_SHARED((DEG_PAD,), jnp.float32))
    scratch += [pltpu.SemaphoreType.DMA] * 8
    return pl.kernel(
        functools.partial(_seg_body, with_deg),
        out_type=out_type,
        mesh=mesh,
        scratch_types=scratch,
    )


_seg_deg = _make_seg(True)
_seg = _make_seg(False)

BLK = 1000
GRID = N // BLK


def _layer_body(h_ref, p_ref, dd_ref, ws_ref, wn_ref, b_ref, o_ref, *, relu):
    agg = p_ref[0] + p_ref[1]
    deg = dd_ref[0] + dd_ref[1]
    rdeg = 1.0 / jnp.maximum(deg, 1.0)
    out = (jnp.dot(h_ref[...], ws_ref[...], preferred_element_type=jnp.float32)
           + jnp.dot(agg * rdeg, wn_ref[...], preferred_element_type=jnp.float32)
           + b_ref[...])
    o_ref[...] = jnp.maximum(out, 0.0) if relu else out


def _head_body(h_ref, p_ref, dd_ref, ws_ref, wn_ref, b_ref, v_ref,
               mw1_ref, mb1_ref, mw2_ref, mb2_ref,
               hw1_ref, hb1_ref, hw2_ref, hb2_ref, o_ref):
    agg = p_ref[0] + p_ref[1]
    deg = dd_ref[0] + dd_ref[1]
    rdeg = 1.0 / jnp.maximum(deg, 1.0)
    h3 = (jnp.dot(h_ref[...], ws_ref[...], preferred_element_type=jnp.float32)
          + jnp.dot(agg * rdeg, wn_ref[...], preferred_element_type=jnp.float32)
          + b_ref[...])
    v = jnp.maximum(
        jnp.dot(v_ref[...], mw1_ref[...], preferred_element_type=jnp.float32)
        + mb1_ref[...], 0.0)
    v = jnp.dot(v, mw2_ref[...], preferred_element_type=jnp.float32) + mb2_ref[...]
    f = h3 + v
    o = jnp.maximum(
        jnp.dot(f, hw1_ref[...], preferred_element_type=jnp.float32)
        + hb1_ref[...], 0.0)
    o_ref[...] = (jnp.dot(o, hw2_ref[...], preferred_element_type=jnp.float32)
                  + hb2_ref[...])


def _full(shape):
    return pl.BlockSpec(shape, lambda i: tuple(0 for _ in shape))


_SPEC_H = pl.BlockSpec((BLK, D), lambda i: (i, 0))
_SPEC_P = pl.BlockSpec((NC, BLK, D), lambda i: (0, i, 0))
_SPEC_DD = pl.BlockSpec((NC, BLK, 1), lambda i: (0, i, 0))


def _tc_layer(h, p, dd, Ws, Wn, b, relu):
    return pl.pallas_call(
        functools.partial(_layer_body, relu=relu),
        grid=(GRID,),
        in_specs=[_SPEC_H, _SPEC_P, _SPEC_DD,
                  _full((D, D)), _full((D, D)), _full((1, D))],
        out_specs=_SPEC_H,
        out_shape=jax.ShapeDtypeStruct((N, D), jnp.float32),
    )(h, p, dd, Ws, Wn, b)


def _tc_head(h, p, dd, Ws, Wn, b, v, mW1, mb1, mW2, mb2, hW1, hb1, hW2, hb2):
    return pl.pallas_call(
        _head_body,
        grid=(GRID,),
        in_specs=[_SPEC_H, _SPEC_P, _SPEC_DD,
                  _full((D, D)), _full((D, D)), _full((1, D)),
                  _full((1, D)),
                  _full((D, D)), _full((1, D)), _full((D, D)), _full((1, D)),
                  _full((D, H)), _full((1, H)), _full((H, 1)), _full((1, 1))],
        out_specs=pl.BlockSpec((BLK, 1), lambda i: (i, 0)),
        out_shape=jax.ShapeDtypeStruct((N, 1), jnp.float32),
    )(h, p, dd, Ws, Wn, b, v, mW1, mb1, mW2, mb2, hW1, hb1, hW2, hb2)


def kernel(x, edge_index, v_node_x, gW0s, gW0n, gb0, gW1s, gW1n, gb1,
           gW2s, gW2n, gb2, mW1, mb1, mW2, mb2, hW1, hb1, hW2, hb2):
    znd = jnp.zeros((NPAD, D), jnp.float32)
    zdeg = jnp.zeros((DEG_PAD,), jnp.float32)
    ones = jnp.ones((CHUNK,), jnp.float32)

    agg0, degp = _seg_deg(x, edge_index, znd, zdeg, ones)
    dd = degp.reshape(NC, DEG_PAD, 1)

    b0 = gb0.reshape(1, D)
    b1 = gb1.reshape(1, D)
    b2 = gb2.reshape(1, D)

    h1 = _tc_layer(x, agg0, dd, gW0s, gW0n, b0, True)
    (agg1,) = _seg(h1, edge_index, znd)
    h2 = _tc_layer(h1, agg1, dd, gW1s, gW1n, b1, True)
    (agg2,) = _seg(h2, edge_index, znd)
    out = _tc_head(h2, agg2, dd, gW2s, gW2n, b2, v_node_x,
                   mW1, mb1.reshape(1, D), mW2, mb2.reshape(1, D),
                   hW1, hb1.reshape(1, H), hW2, hb2.reshape(1, 1))
    return out.reshape(1, N)


# speedup vs baseline: 10.7290x; 1.0268x over previous
import functools

import jax
import jax.numpy as jnp
from jax import lax
from jax.experimental import pallas as pl
from jax.experimental.pallas import tpu as pltpu
from jax.experimental.pallas import tpu_sc as plsc

N = 10000
E = 320000
D = 128
H = 64

NC = 2
NS = 16
NW = NC * NS
CHUNK = 128
NCHT = E // CHUNK
NSLOT = NCHT // NW
LEFT = NCHT - NSLOT * NW
NPAD = 10240
RPT = NPAD // NS
DEG_PAD = NPAD
DPT = DEG_PAD // NS


def _seg_body(with_deg, *refs):
    if with_deg:
        (h_hbm, ei_hbm, znd_hbm, zdeg_hbm, ones_hbm,
         agg_out, deg_out, i0, i1, i2, i3, rows_a, rows_b, ones_v,
         agg_sh, deg_sh, sg0, sg1, ss0, ss1, si0, si1, si2, si3) = refs
    else:
        (h_hbm, ei_hbm, znd_hbm,
         agg_out, i0, i1, i2, i3, rows_a, rows_b, agg_sh,
         sg0, sg1, ss0, ss1, si0, si1, si2, si3) = refs
    c = lax.axis_index("c")
    s = lax.axis_index("s")
    w = s * NC + c

    t0 = s * RPT
    pltpu.sync_copy(znd_hbm.at[pl.ds(t0, RPT)], agg_sh.at[pl.ds(t0, RPT)])
    if with_deg:
        d0 = s * DPT
        pltpu.sync_copy(zdeg_hbm.at[pl.ds(d0, DPT)], deg_sh.at[pl.ds(d0, DPT)])
        pltpu.sync_copy(ones_hbm, ones_v)

    I = [i0, i1, i2, i3]
    R = [rows_a, rows_b]
    GS = [sg0, sg1]
    SS = [ss0, ss1]
    IS = [si0, si1, si2, si3]
    idummy = ei_hbm.at[:, pl.ds(0, CHUNK)]
    rdummy = znd_hbm.at[pl.ds(0, CHUNK)]

    def ifetch(j, k):
        e0 = jnp.minimum(j * NW + w, NCHT - 1) * CHUNK
        pltpu.async_copy(ei_hbm.at[:, pl.ds(e0, CHUNK)], I[k], IS[k])

    def slot(q, j, skip_swait=False):
        p, pn = q % 2, 1 - q % 2
        k, kp, kf = q % 4, (q + 1) % 4, (q + 3) % 4
        pltpu.make_async_copy(rdummy, R[p], GS[p]).wait()
        pltpu.async_copy(R[p], agg_sh.at[I[k].at[1]], SS[p], add=True)
        if with_deg:
            pltpu.async_copy(ones_v, deg_sh.at[I[k].at[1]], SS[p], add=True)
        if not skip_swait:
            pltpu.make_async_copy(rdummy, R[pn], SS[pn]).wait()
            if with_deg:
                pltpu.make_async_copy(ones_hbm, ones_v, SS[pn]).wait()
        ifetch(j + 3, kf)
        pltpu.make_async_copy(idummy, I[kp], IS[kp]).wait()
        pltpu.async_copy(h_hbm.at[I[kp].at[0]],
                         R[pn], GS[pn])

    pltpu.sync_copy(ei_hbm.at[:, pl.ds(w * CHUNK, CHUNK)], i0)
    ifetch(1, 1)
    ifetch(2, 2)
    pltpu.async_copy(h_hbm.at[i0.at[0]], rows_a, sg0)
    plsc.subcore_barrier()

    slot(0, 0, skip_swait=True)
    slot(1, 1)
    slot(2, 2)
    slot(3, 3)

    def group(t, carry):
        j = 4 * t
        slot(0, j)
        slot(1, j + 1)
        slot(2, j + 2)
        slot(3, j + 3)
        return carry

    lax.fori_loop(1, NSLOT // 4, group, 0)
    slot(0, NSLOT - 2)
    slot(1, NSLOT - 1)

    pltpu.make_async_copy(rdummy, rows_a, sg0).wait()

    @pl.when(w < LEFT)
    def _leftover():
        pltpu.async_copy(rows_a, agg_sh.at[i2.at[1]], ss0, add=True)
        if with_deg:
            pltpu.async_copy(ones_v, deg_sh.at[i2.at[1]], ss0, add=True)

    pltpu.make_async_copy(rdummy, rows_b, ss1).wait()
    if with_deg:
        pltpu.make_async_copy(ones_hbm, ones_v, ss1).wait()

    @pl.when(w < LEFT)
    def _drain_leftover():
        pltpu.make_async_copy(rdummy, rows_a, ss0).wait()
        if with_deg:
            pltpu.make_async_copy(ones_hbm, ones_v, ss0).wait()

    pltpu.make_async_copy(idummy, i3, si3).wait()
    pltpu.make_async_copy(idummy, i0, si0).wait()
    plsc.subcore_barrier()

    pltpu.sync_copy(agg_sh.at[pl.ds(t0, RPT)], agg_out.at[c, pl.ds(t0, RPT)])
    if with_deg:
        pltpu.sync_copy(deg_sh.at[pl.ds(d0, DPT)], deg_out.at[c, pl.ds(d0, DPT)])


def _make_seg(with_deg):
    mesh = plsc.VectorSubcoreMesh(core_axis_name="c", subcore_axis_name="s")
    out_type = [jax.ShapeDtypeStruct((NC, NPAD, D), jnp.float32)]
    if with_deg:
        out_type.append(jax.ShapeDtypeStruct((NC, DEG_PAD), jnp.float32))
    scratch = [
        pltpu.VMEM((2, CHUNK), jnp.int32),
        pltpu.VMEM((2, CHUNK), jnp.int32),
        pltpu.VMEM((2, CHUNK), jnp.int32),
        pltpu.VMEM((2, CHUNK), jnp.int32),
        pltpu.VMEM((CHUNK, D), jnp.float32),
        pltpu.VMEM((CHUNK, D), jnp.float32),
    ]
    if with_deg:
        scratch.append(pltpu.VMEM((CHUNK,), jnp.float32))
    scratch += [
        pltpu.VMEM_SHARED((NPAD, D), jnp.float32),
    ]
    if with_deg:
        scratch.append(pltpu.VMEM_SHARED((DEG_PAD,), jnp.float32))
    scratch += [pltpu.SemaphoreType.DMA] * 8
    return pl.kernel(
        functools.partial(_seg_body, with_deg),
        out_type=out_type,
        mesh=mesh,
        scratch_types=scratch,
    )


_seg_deg = _make_seg(True)
_seg = _make_seg(False)

BLK = 2000
GRID = N // BLK


def _layer_body(h_ref, p_ref, dd_ref, ws_ref, wn_ref, b_ref, o_ref, *, relu):
    agg = p_ref[0] + p_ref[1]
    deg = dd_ref[0] + dd_ref[1]
    rdeg = 1.0 / jnp.maximum(deg, 1.0)
    out = (jnp.dot(h_ref[...], ws_ref[...], preferred_element_type=jnp.float32)
           + jnp.dot(agg * rdeg, wn_ref[...], preferred_element_type=jnp.float32)
           + b_ref[...])
    o_ref[...] = jnp.maximum(out, 0.0) if relu else out


def _head_body(h_ref, p_ref, dd_ref, ws_ref, wn_ref, b_ref, v_ref,
               mw1_ref, mb1_ref, mw2_ref, mb2_ref,
               hw1_ref, hb1_ref, hw2_ref, hb2_ref, o_ref):
    agg = p_ref[0] + p_ref[1]
    deg = dd_ref[0] + dd_ref[1]
    rdeg = 1.0 / jnp.maximum(deg, 1.0)
    h3 = (jnp.dot(h_ref[...], ws_ref[...], preferred_element_type=jnp.float32)
          + jnp.dot(agg * rdeg, wn_ref[...], preferred_element_type=jnp.float32)
          + b_ref[...])
    v = jnp.maximum(
        jnp.dot(v_ref[...], mw1_ref[...], preferred_element_type=jnp.float32)
        + mb1_ref[...], 0.0)
    v = jnp.dot(v, mw2_ref[...], preferred_element_type=jnp.float32) + mb2_ref[...]
    f = h3 + v
    o = jnp.maximum(
        jnp.dot(f, hw1_ref[...], preferred_element_type=jnp.float32)
        + hb1_ref[...], 0.0)
    o_ref[...] = (jnp.dot(o, hw2_ref[...], preferred_element_type=jnp.float32)
                  + hb2_ref[...])


def _full(shape):
    return pl.BlockSpec(shape, lambda i: tuple(0 for _ in shape))


_SPEC_H = pl.BlockSpec((BLK, D), lambda i: (i, 0))
_SPEC_P = pl.BlockSpec((NC, BLK, D), lambda i: (0, i, 0))
_SPEC_DD = pl.BlockSpec((NC, BLK, 1), lambda i: (0, i, 0))


def _tc_layer(h, p, dd, Ws, Wn, b, relu):
    return pl.pallas_call(
        functools.partial(_layer_body, relu=relu),
        grid=(GRID,),
        in_specs=[_SPEC_H, _SPEC_P, _SPEC_DD,
                  _full((D, D)), _full((D, D)), _full((1, D))],
        out_specs=_SPEC_H,
        out_shape=jax.ShapeDtypeStruct((N, D), jnp.float32),
    )(h, p, dd, Ws, Wn, b)


def _tc_head(h, p, dd, Ws, Wn, b, v, mW1, mb1, mW2, mb2, hW1, hb1, hW2, hb2):
    return pl.pallas_call(
        _head_body,
        grid=(GRID,),
        in_specs=[_SPEC_H, _SPEC_P, _SPEC_DD,
                  _full((D, D)), _full((D, D)), _full((1, D)),
                  _full((1, D)),
                  _full((D, D)), _full((1, D)), _full((D, D)), _full((1, D)),
                  _full((D, H)), _full((1, H)), _full((H, 1)), _full((1, 1))],
        out_specs=pl.BlockSpec((BLK, 1), lambda i: (i, 0)),
        out_shape=jax.ShapeDtypeStruct((N, 1), jnp.float32),
    )(h, p, dd, Ws, Wn, b, v, mW1, mb1, mW2, mb2, hW1, hb1, hW2, hb2)


def kernel(x, edge_index, v_node_x, gW0s, gW0n, gb0, gW1s, gW1n, gb1,
           gW2s, gW2n, gb2, mW1, mb1, mW2, mb2, hW1, hb1, hW2, hb2):
    znd = jnp.zeros((NPAD, D), jnp.float32)
    zdeg = jnp.zeros((DEG_PAD,), jnp.float32)
    ones = jnp.ones((CHUNK,), jnp.float32)

    agg0, degp = _seg_deg(x, edge_index, znd, zdeg, ones)
    dd = degp.reshape(NC, DEG_PAD, 1)

    b0 = gb0.reshape(1, D)
    b1 = gb1.reshape(1, D)
    b2 = gb2.reshape(1, D)

    h1 = _tc_layer(x, agg0, dd, gW0s, gW0n, b0, True)
    (agg1,) = _seg(h1, edge_index, znd)
    h2 = _tc_layer(h1, agg1, dd, gW1s, gW1n, b1, True)
    (agg2,) = _seg(h2, edge_index, znd)
    out = _tc_head(h2, agg2, dd, gW2s, gW2n, b2, v_node_x,
                   mW1, mb1.reshape(1, D), mW2, mb2.reshape(1, D),
                   hW1, hb1.reshape(1, H), hW2, hb2.reshape(1, 1))
    return out.reshape(1, N)
